# baseline (speedup 1.0000x reference)
import functools

import jax
import jax.numpy as jnp
from jax.experimental import pallas as pl
from jax.experimental.pallas import tpu as pltpu


def _msg_body(uhg_ref, ihg_ref, ue_ref, ie_ref, ge_ref, wt_ref, b_ref, out_ref):
    uh = uhg_ref[...].astype(jnp.bfloat16)
    ih = ihg_ref[...].astype(jnp.bfloat16)
    acc_u = jnp.dot(uh, ue_ref[...], preferred_element_type=jnp.float32)
    acc_i = jnp.dot(ih, ie_ref[...], preferred_element_type=jnp.float32)
    ig = acc_i * ge_ref[...]
    cat = jnp.concatenate([acc_u, acc_i, ig], axis=1).astype(jnp.bfloat16)
    out_ref[...] = (
        jnp.dot(cat, wt_ref[...], preferred_element_type=jnp.float32) + b_ref[...]
    )


def _norm_body(fh_ref, m_ref, out_ref):
    fh = fh_ref[...].astype(jnp.bfloat16)
    out_ref[...] = jnp.dot(fh, m_ref[...], preferred_element_type=jnp.float32)


def _pick_block(n, target):
    if n % target == 0:
        return target
    return n


@jax.jit
def kernel(user_emb, item_emb, group_emb, user_hyper_graph, item_hyper_graph,
           full_hyper, W, b):
    G, U = user_hyper_graph.shape
    _, I = item_hyper_graph.shape
    UI = full_hyper.shape[0]
    D = user_emb.shape[1]

    ue = user_emb.astype(jnp.bfloat16)
    ie = item_emb.astype(jnp.bfloat16)
    wt = W.T.astype(jnp.bfloat16)
    b2 = b.reshape(1, D)

    bm_a = _pick_block(G, 128)
    grid_a = G // bm_a
    msg = pl.pallas_call(
        _msg_body,
        grid=(grid_a,),
        in_specs=[
            pl.BlockSpec((bm_a, U), lambda i: (i, 0)),
            pl.BlockSpec((bm_a, I), lambda i: (i, 0)),
            pl.BlockSpec((U, D), lambda i: (0, 0)),
            pl.BlockSpec((I, D), lambda i: (0, 0)),
            pl.BlockSpec((bm_a, D), lambda i: (i, 0)),
            pl.BlockSpec((3 * D, D), lambda i: (0, 0)),
            pl.BlockSpec((1, D), lambda i: (0, 0)),
        ],
        out_specs=pl.BlockSpec((bm_a, D), lambda i: (i, 0)),
        out_shape=jax.ShapeDtypeStruct((G, D), jnp.float32),
        compiler_params=pltpu.CompilerParams(
            dimension_semantics=("arbitrary",),
            vmem_limit_bytes=60 * 1024 * 1024,
        ),
    )(user_hyper_graph, item_hyper_graph, ue, ie, group_emb, wt, b2)

    msg_bf = msg.astype(jnp.bfloat16)
    bm_b = _pick_block(UI, 1000)
    grid_b = UI // bm_b
    norm_emb = pl.pallas_call(
        _norm_body,
        grid=(grid_b,),
        in_specs=[
            pl.BlockSpec((bm_b, G), lambda i: (i, 0)),
            pl.BlockSpec((G, D), lambda i: (0, 0)),
        ],
        out_specs=pl.BlockSpec((bm_b, D), lambda i: (i, 0)),
        out_shape=jax.ShapeDtypeStruct((UI, D), jnp.float32),
        compiler_params=pltpu.CompilerParams(
            dimension_semantics=("arbitrary",),
            vmem_limit_bytes=60 * 1024 * 1024,
        ),
    )(full_hyper, msg_bf)

    return (norm_emb, msg)


# speedup vs baseline: 2.0209x; 2.0209x over previous
import functools

import jax
import jax.numpy as jnp
from jax import lax
from jax.experimental import pallas as pl
from jax.experimental.pallas import tpu as pltpu


def _dot_t_lhs(a_t, b):
    return lax.dot_general(a_t, b, (((0,), (0,)), ((), ())),
                           preferred_element_type=jnp.float32)


def _dot_t_rhs(a, b_t):
    return lax.dot_general(a, b_t, (((1,), (1,)), ((), ())),
                           preferred_element_type=jnp.float32)


def _msg_body(nsteps, uhgt_ref, ihgt_ref, ue_ref, ie_ref, ge_ref, w_ref, b_ref,
              out_ref, accu_ref, acci_ref):
    k = pl.program_id(0)

    @pl.when(k == 0)
    def _init():
        accu_ref[...] = jnp.zeros_like(accu_ref)
        acci_ref[...] = jnp.zeros_like(acci_ref)

    accu_ref[...] += _dot_t_lhs(uhgt_ref[...], ue_ref[...])
    acci_ref[...] += _dot_t_lhs(ihgt_ref[...], ie_ref[...])

    @pl.when(k == nsteps - 1)
    def _finish():
        acc_u = accu_ref[...]
        acc_i = acci_ref[...]
        ig = acc_i * ge_ref[...]
        d = out_ref.shape[1]
        msg = _dot_t_rhs(acc_u, w_ref[:, :d])
        msg += _dot_t_rhs(acc_i, w_ref[:, d:2 * d])
        msg += _dot_t_rhs(ig, w_ref[:, 2 * d:])
        out_ref[...] = msg + b_ref[...]


def _norm_body(fh_ref, m_ref, out_ref):
    out_ref[...] = jnp.dot(fh_ref[...], m_ref[...],
                           preferred_element_type=jnp.float32)


def _pick_block(n, target):
    if n % target == 0:
        return target
    return n


@jax.jit
def kernel(user_emb, item_emb, group_emb, user_hyper_graph, item_hyper_graph,
           full_hyper, W, b):
    G, U = user_hyper_graph.shape
    _, I = item_hyper_graph.shape
    UI = full_hyper.shape[0]
    D = user_emb.shape[1]

    uhg_t = jnp.swapaxes(user_hyper_graph, 0, 1)
    ihg_t = jnp.swapaxes(item_hyper_graph, 0, 1)
    b2 = b.reshape(1, D)

    bk = _pick_block(U, 400)
    nsteps = U // bk
    msg = pl.pallas_call(
        functools.partial(_msg_body, nsteps),
        grid=(nsteps,),
        in_specs=[
            pl.BlockSpec((bk, G), lambda k: (k, 0)),
            pl.BlockSpec((bk, G), lambda k: (k, 0)),
            pl.BlockSpec((bk, D), lambda k: (k, 0)),
            pl.BlockSpec((bk, D), lambda k: (k, 0)),
            pl.BlockSpec((G, D), lambda k: (0, 0)),
            pl.BlockSpec((D, 3 * D), lambda k: (0, 0)),
            pl.BlockSpec((1, D), lambda k: (0, 0)),
        ],
        out_specs=pl.BlockSpec((G, D), lambda k: (0, 0)),
        out_shape=jax.ShapeDtypeStruct((G, D), jnp.float32),
        scratch_shapes=[
            pltpu.VMEM((G, D), jnp.float32),
            pltpu.VMEM((G, D), jnp.float32),
        ],
        compiler_params=pltpu.CompilerParams(
            dimension_semantics=("arbitrary",),
            vmem_limit_bytes=60 * 1024 * 1024,
        ),
    )(uhg_t, ihg_t, user_emb, item_emb, group_emb, W, b2)

    bm_b = _pick_block(UI, 2000)
    grid_b = UI // bm_b
    norm_emb = pl.pallas_call(
        _norm_body,
        grid=(grid_b,),
        in_specs=[
            pl.BlockSpec((bm_b, G), lambda i: (i, 0)),
            pl.BlockSpec((G, D), lambda i: (0, 0)),
        ],
        out_specs=pl.BlockSpec((bm_b, D), lambda i: (i, 0)),
        out_shape=jax.ShapeDtypeStruct((UI, D), jnp.float32),
        compiler_params=pltpu.CompilerParams(
            dimension_semantics=("arbitrary",),
            vmem_limit_bytes=60 * 1024 * 1024,
        ),
    )(full_hyper, msg)

    return (norm_emb, msg)


# speedup vs baseline: 2.1462x; 1.0620x over previous
import functools

import jax
import jax.numpy as jnp
from jax import lax
from jax.experimental import pallas as pl
from jax.experimental.pallas import tpu as pltpu


def _dot_t_lhs(a_t, b):
    return lax.dot_general(a_t, b, (((0,), (0,)), ((), ())),
                           preferred_element_type=jnp.float32)


def _dot_t_rhs(a, b_t):
    return lax.dot_general(a, b_t, (((1,), (1,)), ((), ())),
                           preferred_element_type=jnp.float32)


def _msg_body(nsteps, uhgt_ref, ihgt_ref, ue_ref, ie_ref, ge_ref, w_ref, b_ref,
              out_ref, acci_ref):
    k = pl.program_id(0)

    @pl.when(k == 0)
    def _init():
        out_ref[...] = jnp.zeros_like(out_ref)
        acci_ref[...] = jnp.zeros_like(acci_ref)

    out_ref[...] += _dot_t_lhs(uhgt_ref[...], ue_ref[...])
    acci_ref[...] += _dot_t_lhs(ihgt_ref[...], ie_ref[...])

    @pl.when(k == nsteps - 1)
    def _finish():
        g, d = out_ref.shape
        cg = 512 if g % 512 == 0 else g
        for c in range(g // cg):
            rows = pl.ds(c * cg, cg)
            acc_u = out_ref[rows, :]
            acc_i = acci_ref[rows, :]
            ig = acc_i * ge_ref[rows, :]
            msg = _dot_t_rhs(acc_u, w_ref[:, :d])
            msg += _dot_t_rhs(acc_i, w_ref[:, d:2 * d])
            msg += _dot_t_rhs(ig, w_ref[:, 2 * d:])
            out_ref[rows, :] = msg + b_ref[...]


def _norm_body(fh_ref, m_ref, out_ref):
    out_ref[...] = jnp.dot(fh_ref[...], m_ref[...],
                           preferred_element_type=jnp.float32)


def _pick_block(n, target):
    if n % target == 0:
        return target
    return n


@jax.jit
def kernel(user_emb, item_emb, group_emb, user_hyper_graph, item_hyper_graph,
           full_hyper, W, b):
    G, U = user_hyper_graph.shape
    _, I = item_hyper_graph.shape
    UI = full_hyper.shape[0]
    D = user_emb.shape[1]

    uhg_t = jnp.swapaxes(user_hyper_graph, 0, 1)
    ihg_t = jnp.swapaxes(item_hyper_graph, 0, 1)
    b2 = b.reshape(1, D)

    bk = _pick_block(U, 1000)
    nsteps = U // bk
    msg = pl.pallas_call(
        functools.partial(_msg_body, nsteps),
        grid=(nsteps,),
        in_specs=[
            pl.BlockSpec((bk, G), lambda k: (k, 0)),
            pl.BlockSpec((bk, G), lambda k: (k, 0)),
            pl.BlockSpec((bk, D), lambda k: (k, 0)),
            pl.BlockSpec((bk, D), lambda k: (k, 0)),
            pl.BlockSpec((G, D), lambda k: (0, 0)),
            pl.BlockSpec((D, 3 * D), lambda k: (0, 0)),
            pl.BlockSpec((1, D), lambda k: (0, 0)),
        ],
        out_specs=pl.BlockSpec((G, D), lambda k: (0, 0)),
        out_shape=jax.ShapeDtypeStruct((G, D), jnp.float32),
        scratch_shapes=[
            pltpu.VMEM((G, D), jnp.float32),
        ],
        compiler_params=pltpu.CompilerParams(
            dimension_semantics=("arbitrary",),
            vmem_limit_bytes=60 * 1024 * 1024,
        ),
    )(uhg_t, ihg_t, user_emb, item_emb, group_emb, W, b2)

    bm_b = _pick_block(UI, 2000)
    grid_b = UI // bm_b
    norm_emb = pl.pallas_call(
        _norm_body,
        grid=(grid_b,),
        in_specs=[
            pl.BlockSpec((bm_b, G), lambda i: (i, 0)),
            pl.BlockSpec((G, D), lambda i: (0, 0)),
        ],
        out_specs=pl.BlockSpec((bm_b, D), lambda i: (i, 0)),
        out_shape=jax.ShapeDtypeStruct((UI, D), jnp.float32),
        compiler_params=pltpu.CompilerParams(
            dimension_semantics=("arbitrary",),
            vmem_limit_bytes=60 * 1024 * 1024,
        ),
    )(full_hyper, msg)

    return (norm_emb, msg)
